# baseline (speedup 1.0000x reference)
import functools

import jax
import jax.numpy as jnp
from jax import lax
from jax.experimental import pallas as pl
from jax.experimental.pallas import tpu as pltpu
from jax.experimental.pallas import tpu_sc as plsc

B, S, D, V = 4096, 200, 64, 202
TOT = B * S
IW = 128
NROWS = TOT // IW
NW = 32
RPW = NROWS // NW
K = 4
NCH = RPW // K
CH = K * IW
NPAIR = NCH // 2


def _sc_lookup(idx_f, idx_r, table):
    mesh = plsc.VectorSubcoreMesh(core_axis_name="c", subcore_axis_name="s")

    @functools.partial(
        pl.kernel,
        mesh=mesh,
        out_type=[jax.ShapeDtypeStruct((TOT, D), jnp.float32),
                  jax.ShapeDtypeStruct((TOT, D), jnp.float32)],
        compiler_params=pltpu.CompilerParams(use_tc_tiling_on_sc=False),
        scratch_types=[
            pltpu.VMEM((RPW, IW), jnp.int32),
            pltpu.VMEM((CH, D), jnp.float32),
            pltpu.VMEM((CH, D), jnp.float32),
            pltpu.SemaphoreType.DMA,
            pltpu.SemaphoreType.DMA,
            pltpu.SemaphoreType.DMA,
            pltpu.SemaphoreType.DMA,
        ],
    )
    def run(idx_f_hbm, idx_r_hbm, table_hbm, out_f_hbm, out_r_hbm,
            idx_all, rows0, rows1, gsem0, gsem1, wsem0, wsem1):
        wid = lax.axis_index("s") * 2 + lax.axis_index("c")
        base_irow = wid * RPW
        base_out = wid * RPW * IW

        def fire(c, rows, gsem):
            for j in range(K):
                pltpu.async_copy(table_hbm.at[idx_all.at[c * K + j]],
                                 rows.at[pl.ds(j * IW, IW)], gsem)

        def drain(out_hbm, rows, sem):
            pltpu.make_async_copy(out_hbm.at[pl.ds(0, CH)], rows, sem).wait()

        for idx_hbm, out_hbm in ((idx_f_hbm, out_f_hbm),
                                 (idx_r_hbm, out_r_hbm)):
            pltpu.sync_copy(idx_hbm.at[pl.ds(base_irow, RPW)], idx_all)
            fire(0, rows0, gsem0)
            fire(1, rows1, gsem1)

            def body(g, carry, out_hbm=out_hbm):
                c0 = 2 * g
                drain(out_hbm, rows0, gsem0)
                pltpu.async_copy(
                    rows0, out_hbm.at[pl.ds(base_out + c0 * CH, CH)], wsem0)
                drain(out_hbm, rows1, gsem1)
                pltpu.async_copy(
                    rows1, out_hbm.at[pl.ds(base_out + (c0 + 1) * CH, CH)],
                    wsem1)

                @pl.when(g + 1 < NPAIR)
                def _():
                    drain(out_hbm, rows0, wsem0)
                    fire(c0 + 2, rows0, gsem0)
                    drain(out_hbm, rows1, wsem1)
                    fire(c0 + 3, rows1, gsem1)

                return carry

            lax.fori_loop(0, NPAIR, body, 0)
            drain(out_hbm, rows0, wsem0)
            drain(out_hbm, rows1, wsem1)

    return run(idx_f, idx_r, table)


NREP = 256


def kernel(position_index, reversed_position_index, table):
    rep = (V * (jnp.arange(TOT, dtype=jnp.int32) % NREP)).reshape(NROWS, IW)
    idx_f = position_index.reshape(NROWS, IW) + rep
    idx_r = reversed_position_index.reshape(NROWS, IW) + rep
    table_rep = jnp.tile(table, (NREP, 1))
    out_f, out_r = _sc_lookup(idx_f, idx_r, table_rep)
    return (out_f.reshape(B, S, D), out_r.reshape(B, S, D))


# speedup vs baseline: 1.0751x; 1.0751x over previous
import functools

import jax
import jax.numpy as jnp
from jax import lax
from jax.experimental import pallas as pl
from jax.experimental.pallas import tpu as pltpu
from jax.experimental.pallas import tpu_sc as plsc

B, S, D, V = 4096, 200, 64, 202
TOT = B * S
IW = 128
NROWS = TOT // IW
NW = 32
RPW = NROWS // NW
K = 4
NCH = RPW // K
CH = K * IW
NPAIR = NCH // 2


def _sc_lookup(idx_f, idx_r, table):
    mesh = plsc.VectorSubcoreMesh(core_axis_name="c", subcore_axis_name="s")

    @functools.partial(
        pl.kernel,
        mesh=mesh,
        out_type=[jax.ShapeDtypeStruct((TOT, D), jnp.float32),
                  jax.ShapeDtypeStruct((TOT, D), jnp.float32)],
        compiler_params=pltpu.CompilerParams(use_tc_tiling_on_sc=False),
        scratch_types=[
            pltpu.VMEM((RPW, IW), jnp.int32),
            pltpu.VMEM((CH, D), jnp.float32),
            pltpu.VMEM((CH, D), jnp.float32),
            pltpu.VMEM_SHARED((202 * 32, D), jnp.float32),
            pltpu.SemaphoreType.DMA,
            pltpu.SemaphoreType.DMA,
            pltpu.SemaphoreType.DMA,
            pltpu.SemaphoreType.DMA,
        ],
    )
    def run(idx_f_hbm, idx_r_hbm, table_hbm, out_f_hbm, out_r_hbm,
            idx_all, rows0, rows1, table_sh, gsem0, gsem1, wsem0, wsem1):
        wid = lax.axis_index("s") * 2 + lax.axis_index("c")
        base_irow = wid * RPW
        base_out = wid * RPW * IW

        @pl.when(lax.axis_index("s") == 0)
        def _():
            pltpu.sync_copy(table_hbm, table_sh)

        plsc.subcore_barrier()

        def fire(c, rows, gsem):
            for j in range(K):
                pltpu.async_copy(table_sh.at[idx_all.at[c * K + j]],
                                 rows.at[pl.ds(j * IW, IW)], gsem)

        def drain(out_hbm, rows, sem):
            pltpu.make_async_copy(out_hbm.at[pl.ds(0, CH)], rows, sem).wait()

        for idx_hbm, out_hbm in ((idx_f_hbm, out_f_hbm),
                                 (idx_r_hbm, out_r_hbm)):
            pltpu.sync_copy(idx_hbm.at[pl.ds(base_irow, RPW)], idx_all)
            fire(0, rows0, gsem0)
            fire(1, rows1, gsem1)

            def body(g, carry, out_hbm=out_hbm):
                c0 = 2 * g
                drain(out_hbm, rows0, gsem0)
                pltpu.async_copy(
                    rows0, out_hbm.at[pl.ds(base_out + c0 * CH, CH)], wsem0)
                drain(out_hbm, rows1, gsem1)
                pltpu.async_copy(
                    rows1, out_hbm.at[pl.ds(base_out + (c0 + 1) * CH, CH)],
                    wsem1)

                @pl.when(g + 1 < NPAIR)
                def _():
                    drain(out_hbm, rows0, wsem0)
                    fire(c0 + 2, rows0, gsem0)
                    drain(out_hbm, rows1, wsem1)
                    fire(c0 + 3, rows1, gsem1)

                return carry

            lax.fori_loop(0, NPAIR, body, 0)
            drain(out_hbm, rows0, wsem0)
            drain(out_hbm, rows1, wsem1)

    return run(idx_f, idx_r, table)


NREP = 32


def kernel(position_index, reversed_position_index, table):
    rep = (V * (jnp.arange(TOT, dtype=jnp.int32) % NREP)).reshape(NROWS, IW)
    idx_f = position_index.reshape(NROWS, IW) + rep
    idx_r = reversed_position_index.reshape(NROWS, IW) + rep
    table_rep = jnp.tile(table, (NREP, 1))
    out_f, out_r = _sc_lookup(idx_f, idx_r, table_rep)
    return (out_f.reshape(B, S, D), out_r.reshape(B, S, D))


# speedup vs baseline: 1.0753x; 1.0002x over previous
import functools

import jax
import jax.numpy as jnp
from jax import lax
from jax.experimental import pallas as pl
from jax.experimental.pallas import tpu as pltpu
from jax.experimental.pallas import tpu_sc as plsc

B, S, D, V = 4096, 200, 64, 202
TOT = B * S
IW = 128
NROWS = TOT // IW
NW = 32
RPW = NROWS // NW
K = 4
NCH = RPW // K
CH = K * IW
NPAIR = NCH // 2


def _sc_lookup(idx_f, idx_r, table, table_rep):
    mesh = plsc.VectorSubcoreMesh(core_axis_name="c", subcore_axis_name="s")

    @functools.partial(
        pl.kernel,
        mesh=mesh,
        out_type=[jax.ShapeDtypeStruct((TOT, D), jnp.float32),
                  jax.ShapeDtypeStruct((TOT, D), jnp.float32)],
        compiler_params=pltpu.CompilerParams(use_tc_tiling_on_sc=False),
        scratch_types=[
            pltpu.VMEM((RPW, IW), jnp.int32),
            pltpu.VMEM((CH, D), jnp.float32),
            pltpu.VMEM((CH, D), jnp.float32),
            pltpu.VMEM_SHARED((V, D), jnp.float32),
            pltpu.SemaphoreType.DMA,
            pltpu.SemaphoreType.DMA,
            pltpu.SemaphoreType.DMA,
            pltpu.SemaphoreType.DMA,
        ],
    )
    def run(idx_f_hbm, idx_r_hbm, table_hbm, trep_hbm, out_f_hbm, out_r_hbm,
            idx_all, rows0, rows1, table_sh, gsem0, gsem1, wsem0, wsem1):
        wid = lax.axis_index("s") * 2 + lax.axis_index("c")
        base_irow = wid * RPW
        base_out = wid * RPW * IW

        @pl.when(lax.axis_index("s") == 0)
        def _():
            pltpu.sync_copy(table_hbm, table_sh)

        plsc.subcore_barrier()

        def fire(c, rows, gsem, from_spmem):
            tab = table_sh if from_spmem else trep_hbm
            for j in range(K):
                pltpu.async_copy(tab.at[idx_all.at[c * K + j]],
                                 rows.at[pl.ds(j * IW, IW)], gsem)

        def drain(out_hbm, rows, sem):
            pltpu.make_async_copy(out_hbm.at[pl.ds(0, CH)], rows, sem).wait()

        for idx_hbm, out_hbm in ((idx_f_hbm, out_f_hbm),
                                 (idx_r_hbm, out_r_hbm)):
            pltpu.sync_copy(idx_hbm.at[pl.ds(base_irow, RPW)], idx_all)
            fire(0, rows0, gsem0, True)
            fire(1, rows1, gsem1, False)

            def body(g, carry, out_hbm=out_hbm):
                c0 = 2 * g
                drain(out_hbm, rows0, gsem0)
                pltpu.async_copy(
                    rows0, out_hbm.at[pl.ds(base_out + c0 * CH, CH)], wsem0)
                drain(out_hbm, rows1, gsem1)
                pltpu.async_copy(
                    rows1, out_hbm.at[pl.ds(base_out + (c0 + 1) * CH, CH)],
                    wsem1)

                @pl.when(g + 1 < NPAIR)
                def _():
                    drain(out_hbm, rows0, wsem0)
                    fire(c0 + 2, rows0, gsem0, True)
                    drain(out_hbm, rows1, wsem1)
                    fire(c0 + 3, rows1, gsem1, False)

                return carry

            lax.fori_loop(0, NPAIR, body, 0)
            drain(out_hbm, rows0, wsem0)
            drain(out_hbm, rows1, wsem1)

    return run(idx_f, idx_r, table, table_rep)


NREP = 64


def kernel(position_index, reversed_position_index, table):
    odd_chunk_row = ((jnp.arange(NROWS, dtype=jnp.int32) // K) % 2 == 1)
    rep = V * (jnp.arange(TOT, dtype=jnp.int32) % NREP).reshape(NROWS, IW)
    rep = jnp.where(odd_chunk_row[:, None], rep, 0)
    idx_f = position_index.reshape(NROWS, IW) + rep
    idx_r = reversed_position_index.reshape(NROWS, IW) + rep
    table_rep = jnp.tile(table, (NREP, 1))
    out_f, out_r = _sc_lookup(idx_f, idx_r, table, table_rep)
    return (out_f.reshape(B, S, D), out_r.reshape(B, S, D))
